# baseline (speedup 1.0000x reference)
import jax
import jax.numpy as jnp
from jax.experimental import pallas as pl
from jax.experimental.pallas import tpu as pltpu

_PAIRS = [(0, 12345), (1, 987654), (2, 555555), (0, 42), (2, 999999), (1, 31337), (0, 777777), (1, 0), (2, 123456), (0, 654321), (1, 222222), (2, 888888), (0, 333333), (1, 444444), (2, 55555), (0, 99999), (1, 666666), (2, 7), (0, 500000), (1, 123), (2, 345678), (0, 876543), (1, 999998), (2, 111111), (0, 1), (1, 750000), (2, 250000), (0, 424242), (1, 313131), (2, 202020), (0, 909090), (1, 818181)]

_OUT_N = len(_PAIRS)
_W = 128
_VALUES_N = 1000000
_TAIL_START = (_VALUES_N // _W) * _W
_TAIL_BLOCK = _VALUES_N // _W

_TAIL_LAYERS = sorted({l for (l, o) in _PAIRS if o >= _TAIL_START})

_SLOTS = {}
for _l, _o in _PAIRS:
    if _o < _TAIL_START:
        _SLOTS.setdefault((_l, (_o // _W) * _W), len(_SLOTS))
_N_SLOTS = len(_SLOTS)


def _gather_body(*refs):
    v0, v1, v2 = refs[:3]
    tails = refs[3:3 + len(_TAIL_LAYERS)]
    out_ref = refs[3 + len(_TAIL_LAYERS)]
    win_ref, sem = refs[4 + len(_TAIL_LAYERS):]

    values = (v0, v1, v2)
    tail_of = dict(zip(_TAIL_LAYERS, tails))
    copies = [
        pltpu.make_async_copy(values[layer].at[pl.ds(start, _W)],
                              win_ref.at[slot], sem.at[slot])
        for (layer, start), slot in _SLOTS.items()
    ]
    for c in copies:
        c.start()
    for i, (layer, ordinal) in enumerate(_PAIRS):
        if ordinal >= _TAIL_START:
            out_ref[i] = tail_of[layer][ordinal % _W]
    for (key, slot), c in zip(_SLOTS.items(), copies):
        c.wait()
        for i, (layer, ordinal) in enumerate(_PAIRS):
            if (ordinal < _TAIL_START
                    and _SLOTS[(layer, (ordinal // _W) * _W)] == slot):
                out_ref[i] = win_ref[slot, ordinal % _W]


def kernel(layer_values_0, layer_values_1, layer_values_2):
    values = (layer_values_0, layer_values_1, layer_values_2)
    tail_inputs = [values[l] for l in _TAIL_LAYERS]
    return pl.pallas_call(
        _gather_body,
        grid=(1,),
        in_specs=[pl.BlockSpec(memory_space=pl.ANY)] * 3
        + [pl.BlockSpec((_W,), lambda g: (_TAIL_BLOCK,),
                        memory_space=pltpu.SMEM)] * len(_TAIL_LAYERS),
        out_specs=pl.BlockSpec((_OUT_N,), lambda g: (0,),
                               memory_space=pltpu.SMEM),
        out_shape=jax.ShapeDtypeStruct((_OUT_N,), jnp.float32),
        scratch_shapes=[
            pltpu.SMEM((_N_SLOTS, _W), jnp.float32),
            pltpu.SemaphoreType.DMA((_N_SLOTS,)),
        ],
    )(*values, *tail_inputs)


# speedup vs baseline: 1.1995x; 1.1995x over previous
import jax
import jax.numpy as jnp
from jax.experimental import pallas as pl
from jax.experimental.pallas import tpu as pltpu

_PAIRS = [(0, 12345), (1, 987654), (2, 555555), (0, 42), (2, 999999), (1, 31337), (0, 777777), (1, 0), (2, 123456), (0, 654321), (1, 222222), (2, 888888), (0, 333333), (1, 444444), (2, 55555), (0, 99999), (1, 666666), (2, 7), (0, 500000), (1, 123), (2, 345678), (0, 876543), (1, 999998), (2, 111111), (0, 1), (1, 750000), (2, 250000), (0, 424242), (1, 313131), (2, 202020), (0, 909090), (1, 818181)]

_OUT_N = len(_PAIRS)
_W = 128
_VALUES_N = 1000000
_TAIL_START = (_VALUES_N // _W) * _W
_TAIL_BLOCK = _VALUES_N // _W

_TAIL_LAYERS = sorted({l for (l, o) in _PAIRS if o >= _TAIL_START})

_SLOTS = {}
for _l, _o in _PAIRS:
    if _o < _TAIL_START:
        _SLOTS.setdefault((_l, (_o // _W) * _W), len(_SLOTS))
_N_SLOTS = len(_SLOTS)


def _gather_body(*refs):
    v0, v1, v2 = refs[:3]
    tails = refs[3:3 + len(_TAIL_LAYERS)]
    out_ref = refs[3 + len(_TAIL_LAYERS)]
    win_ref, sem = refs[4 + len(_TAIL_LAYERS):]

    values = (v0, v1, v2)
    tail_of = dict(zip(_TAIL_LAYERS, tails))
    copies = [
        pltpu.make_async_copy(values[layer].at[pl.ds(start, _W)],
                              win_ref.at[slot], sem)
        for (layer, start), slot in _SLOTS.items()
    ]
    for c in copies:
        c.start()
    for i, (layer, ordinal) in enumerate(_PAIRS):
        if ordinal >= _TAIL_START:
            out_ref[i] = tail_of[layer][ordinal % _W]
    pltpu.make_async_copy(win_ref, win_ref, sem).wait()
    for i, (layer, ordinal) in enumerate(_PAIRS):
        if ordinal < _TAIL_START:
            slot = _SLOTS[(layer, (ordinal // _W) * _W)]
            out_ref[i] = win_ref[slot, ordinal % _W]


def kernel(layer_values_0, layer_values_1, layer_values_2):
    values = (layer_values_0, layer_values_1, layer_values_2)
    tail_inputs = [values[l] for l in _TAIL_LAYERS]
    return pl.pallas_call(
        _gather_body,
        grid=(1,),
        in_specs=[pl.BlockSpec(memory_space=pl.ANY)] * 3
        + [pl.BlockSpec((_W,), lambda g: (_TAIL_BLOCK,),
                        memory_space=pltpu.SMEM)] * len(_TAIL_LAYERS),
        out_specs=pl.BlockSpec((_OUT_N,), lambda g: (0,),
                               memory_space=pltpu.SMEM),
        out_shape=jax.ShapeDtypeStruct((_OUT_N,), jnp.float32),
        scratch_shapes=[
            pltpu.SMEM((_N_SLOTS, _W), jnp.float32),
            pltpu.SemaphoreType.DMA,
        ],
    )(*values, *tail_inputs)
